# baseline (speedup 1.0000x reference)
import functools

import jax
import jax.numpy as jnp
from jax import lax
from jax.experimental import pallas as pl
from jax.experimental.pallas import tpu as pltpu
from jax.experimental.pallas import tpu_sc as plsc

VOCAB = 1000000
EMBED = 32
BATCH = 16384
NEG = 20

NUM_CORES = 2
NUM_SUBCORES = 16
NW = NUM_CORES * NUM_SUBCORES
BPW = BATCH // NW
BC = 16
NBLK = BPW // BC
NROWS_BLK = BC * NEG
NEG_CHUNKS = ((0, 128), (128, 128), (256, 64))

S = 250880
GW = 5120
SBG = S // GW
CLAMP_J = (VOCAB + GW - 1) // GW - 1

_SC_PARAMS = pltpu.CompilerParams(needs_layout_passes=False)


def _pack_body(w0, w1, w2, w3, ow):
    ow[...] = jnp.concatenate(
        [w0[...], w1[...], w2[...], w3[...]], axis=0).T


def _pack_one(W):
    wt = W.T

    def in_spec(q):
        return pl.BlockSpec(
            (EMBED, GW),
            lambda i, q=q: (0, jnp.minimum(q * SBG + i, CLAMP_J)))

    return pl.pallas_call(
        _pack_body,
        grid=(SBG,),
        in_specs=[in_spec(q) for q in range(4)],
        out_specs=pl.BlockSpec((GW, 128), lambda i: (i, 0)),
        out_shape=jax.ShapeDtypeStruct((S, 128), jnp.float32),
    )(wt, wt, wt, wt)


def _neg_body(wp_u, nsup_h, ncb_h, nsum_h,
              nsup_v, ncb_v, nrow0, nrow1, nsum_v, sem):
    cid = lax.axis_index("c")
    sid = lax.axis_index("s")
    wid = sid * NUM_CORES + cid

    pltpu.sync_copy(nsup_h.at[wid], nsup_v)
    pltpu.sync_copy(ncb_h.at[wid], ncb_v)

    iota = lax.iota(jnp.int32, 16)
    nrows0 = iota * NEG
    bufs = (nrow0, nrow1)

    def issue(g, nrow_b):
        for off, ln in NEG_CHUNKS:
            pltpu.async_copy(
                wp_u.at[nsup_v.at[pl.ds(g * NROWS_BLK + off, ln)]],
                nrow_b.at[pl.ds(off, ln)], sem)

    def drain(nrow_b):
        for off, ln in NEG_CHUNKS:
            pltpu.make_async_copy(wp_u.at[pl.ds(0, ln)],
                                  nrow_b.at[pl.ds(off, ln)],
                                  sem).wait()

    def compute(blk, nrow_b):
        out_base = (blk * BC + iota) * EMBED

        @pl.loop(0, EMBED)
        def _dim(d):
            s = jnp.zeros((16,), jnp.float32)
            for k in range(NEG):
                cb_nk = ncb_v[pl.ds(k * BPW + blk * BC, 16)]
                s = s + plsc.load_gather(nrow_b, [nrows0 + k, cb_nk + d])
            plsc.store_scatter(nsum_v, [out_base + d], s)

    issue(0, bufs[0])

    @pl.loop(0, NBLK, step=2)
    def _pair(blk):
        for b in (0, 1):
            g = blk + b

            @pl.when(g + 1 < NBLK)
            def _():
                issue(g + 1, bufs[1 - b])

            drain(bufs[b])
            compute(g, bufs[b])

    pltpu.sync_copy(nsum_v, nsum_h.at[wid])


def _make_neg_kernel():
    mesh = plsc.VectorSubcoreMesh(core_axis_name="c", subcore_axis_name="s")
    return pl.kernel(
        _neg_body,
        out_type=jax.ShapeDtypeStruct((NW, BPW * EMBED), jnp.float32),
        mesh=mesh,
        scratch_types=(
            pltpu.VMEM((BPW * NEG,), jnp.int32),
            pltpu.VMEM((BPW * NEG,), jnp.int32),
            pltpu.VMEM((NROWS_BLK, 128), jnp.float32),
            pltpu.VMEM((NROWS_BLK, 128), jnp.float32),
            pltpu.VMEM((BPW * EMBED,), jnp.float32),
            pltpu.SemaphoreType.DMA,
        ),
        compiler_params=_SC_PARAMS,
    )


def _dot_body(wp_w, wp_u, isup_h, tsup_h, icb_h, tcb_h, nsum_h,
              pos_h, negl_h,
              isup_v, tsup_v, icb_v, tcb_v, nsum_v,
              emb0, ctx0, emb1, ctx1, pos_v, negl_v, sem):
    cid = lax.axis_index("c")
    sid = lax.axis_index("s")
    wid = sid * NUM_CORES + cid

    pltpu.sync_copy(isup_h.at[wid], isup_v)
    pltpu.sync_copy(tsup_h.at[wid], tsup_v)
    pltpu.sync_copy(icb_h.at[wid], icb_v)
    pltpu.sync_copy(tcb_h.at[wid], tcb_v)
    pltpu.sync_copy(nsum_h.at[wid], nsum_v)

    iota = lax.iota(jnp.int32, 16)
    bufs = ((emb0, ctx0), (emb1, ctx1))

    def issue(g, emb_b, ctx_b):
        pltpu.async_copy(wp_w.at[isup_v.at[pl.ds(g * BC, BC)]], emb_b, sem)
        pltpu.async_copy(wp_u.at[tsup_v.at[pl.ds(g * BC, BC)]], ctx_b, sem)

    def drain(emb_b, ctx_b):
        pltpu.make_async_copy(wp_w.at[pl.ds(0, BC)], emb_b, sem).wait()
        pltpu.make_async_copy(wp_u.at[pl.ds(0, BC)], ctx_b, sem).wait()

    def compute(blk, emb_b, ctx_b):
        cb_e = icb_v[pl.ds(blk * BC, 16)]
        cb_c = tcb_v[pl.ds(blk * BC, 16)]
        ns_base = (blk * BC + iota) * EMBED
        zero = jnp.zeros((16,), jnp.float32)

        @pl.loop(0, EMBED, init_carry=(zero, zero))
        def _dim(d, carry):
            acc_p, acc_n = carry
            e = plsc.load_gather(emb_b, [iota, cb_e + d])
            c = plsc.load_gather(ctx_b, [iota, cb_c + d])
            ns = plsc.load_gather(nsum_v, [ns_base + d])
            return acc_p + e * c, acc_n + e * ns

        acc_p, acc_n = _dim
        pos_v[pl.ds(blk * BC, 16)] = acc_p
        negl_v[pl.ds(blk * BC, 16)] = -acc_n

    issue(0, *bufs[0])

    @pl.loop(0, NBLK, step=2)
    def _pair(blk):
        for b in (0, 1):
            g = blk + b

            @pl.when(g + 1 < NBLK)
            def _():
                issue(g + 1, *bufs[1 - b])

            drain(*bufs[b])
            compute(g, *bufs[b])

    pltpu.sync_copy(pos_v, pos_h.at[wid])
    pltpu.sync_copy(negl_v, negl_h.at[wid])


def _make_dot_kernel():
    mesh = plsc.VectorSubcoreMesh(core_axis_name="c", subcore_axis_name="s")
    return pl.kernel(
        _dot_body,
        out_type=(
            jax.ShapeDtypeStruct((NW, BPW), jnp.float32),
            jax.ShapeDtypeStruct((NW, BPW), jnp.float32),
        ),
        mesh=mesh,
        scratch_types=(
            pltpu.VMEM((BPW,), jnp.int32),
            pltpu.VMEM((BPW,), jnp.int32),
            pltpu.VMEM((BPW,), jnp.int32),
            pltpu.VMEM((BPW,), jnp.int32),
            pltpu.VMEM((BPW * EMBED,), jnp.float32),
            pltpu.VMEM((BC, 128), jnp.float32),
            pltpu.VMEM((BC, 128), jnp.float32),
            pltpu.VMEM((BC, 128), jnp.float32),
            pltpu.VMEM((BC, 128), jnp.float32),
            pltpu.VMEM((BPW,), jnp.float32),
            pltpu.VMEM((BPW,), jnp.float32),
            pltpu.SemaphoreType.DMA,
        ),
        compiler_params=_SC_PARAMS,
    )


def _loss_body(pos_ref, negl_ref, out_ref):
    def logsig(x):
        return jnp.minimum(x, 0.0) - jnp.log1p(jnp.exp(-jnp.abs(x)))

    total = jnp.sum(logsig(pos_ref[...])) + jnp.sum(logsig(negl_ref[...]))
    out_ref[0, 0] = -total / BATCH


def _split_idx(v):
    q = v // S
    return (v - q * S).astype(jnp.int32), (q * 32).astype(jnp.int32)


@jax.jit
def kernel(inputs, targets, neg_samples, W_w, W_u):
    wp_u = _pack_one(W_u)
    wp_w = _pack_one(W_w)

    isup, icb = _split_idx(inputs.astype(jnp.int32).reshape(BATCH))
    tsup, tcb = _split_idx(targets.astype(jnp.int32).reshape(BATCH))
    nsup, ncb = _split_idx(neg_samples.astype(jnp.int32))

    isup_h = isup.reshape(NW, BPW)
    tsup_h = tsup.reshape(NW, BPW)
    nsup_h = nsup.reshape(NW, BPW * NEG)
    icb_h = icb.reshape(NW, BPW)
    tcb_h = tcb.reshape(NW, BPW)
    ncb_h = ncb.reshape(NW, BPW, NEG).transpose(0, 2, 1).reshape(NW, BPW * NEG)

    nsum = _make_neg_kernel()(wp_u, nsup_h, ncb_h)
    pos, negl = _make_dot_kernel()(
        wp_w, wp_u, isup_h, tsup_h, icb_h, tcb_h, nsum)

    loss = pl.pallas_call(
        _loss_body,
        out_shape=jax.ShapeDtypeStruct((1, 1), jnp.float32),
        out_specs=pl.BlockSpec(memory_space=pltpu.SMEM),
    )(pos.reshape(128, 128), negl.reshape(128, 128))
    return loss[0, 0]


# speedup vs baseline: 1.0302x; 1.0302x over previous
import functools

import jax
import jax.numpy as jnp
from jax import lax
from jax.experimental import pallas as pl
from jax.experimental.pallas import tpu as pltpu
from jax.experimental.pallas import tpu_sc as plsc

VOCAB = 1000000
EMBED = 32
BATCH = 16384
NEG = 20

NUM_CORES = 2
NUM_SUBCORES = 16
NW = NUM_CORES * NUM_SUBCORES
BPW = BATCH // NW
BC = 16
NBLK = BPW // BC
NROWS_BLK = BC * NEG
NEG_CHUNKS = ((0, 128), (128, 128), (256, 64))

S = 250880
GW = 5120
SBG = S // GW
CLAMP_J = (VOCAB + GW - 1) // GW - 1

_SC_PARAMS = pltpu.CompilerParams(needs_layout_passes=False)


def _pack_body(w0, w1, w2, w3, ow):
    ow[...] = jnp.concatenate(
        [w0[...], w1[...], w2[...], w3[...]], axis=0).T


def _pack_one(W):
    wt = W.T

    def in_spec(q):
        return pl.BlockSpec(
            (EMBED, GW),
            lambda i, q=q: (0, jnp.minimum(q * SBG + i, CLAMP_J)))

    return pl.pallas_call(
        _pack_body,
        grid=(SBG,),
        in_specs=[in_spec(q) for q in range(4)],
        out_specs=pl.BlockSpec((GW, 128), lambda i: (i, 0)),
        out_shape=jax.ShapeDtypeStruct((S, 128), jnp.float32),
    )(wt, wt, wt, wt)


def _neg_body(wp_u, nsup_h, ncb_h, nsum_h,
              nsup_v, ncb_v, nrow0, nrow1, nsum_v, sem):
    cid = lax.axis_index("c")
    sid = lax.axis_index("s")
    wid = sid * NUM_CORES + cid

    pltpu.sync_copy(nsup_h.at[wid], nsup_v)
    pltpu.sync_copy(ncb_h.at[wid], ncb_v)

    iota = lax.iota(jnp.int32, 16)
    nrows0 = iota * NEG
    bufs = (nrow0, nrow1)

    def issue(g, nrow_b):
        for off, ln in NEG_CHUNKS:
            pltpu.async_copy(
                wp_u.at[nsup_v.at[pl.ds(g * NROWS_BLK + off, ln)]],
                nrow_b.at[pl.ds(off, ln)], sem)

    def drain(nrow_b):
        for off, ln in NEG_CHUNKS:
            pltpu.make_async_copy(wp_u.at[pl.ds(0, ln)],
                                  nrow_b.at[pl.ds(off, ln)],
                                  sem).wait()

    def compute(blk, nrow_b):
        @pl.loop(0, EMBED)
        def _dim(d):
            s = jnp.zeros((16,), jnp.float32)
            for k in range(NEG):
                cb_nk = ncb_v[pl.ds(k * BPW + blk * BC, 16)]
                s = s + plsc.load_gather(nrow_b, [nrows0 + k, cb_nk + d])
            nsum_v[pl.ds(d * BPW + blk * BC, 16)] = s

    issue(0, bufs[0])

    @pl.loop(0, NBLK, step=2)
    def _pair(blk):
        for b in (0, 1):
            g = blk + b

            @pl.when(g + 1 < NBLK)
            def _():
                issue(g + 1, bufs[1 - b])

            drain(bufs[b])
            compute(g, bufs[b])

    pltpu.sync_copy(nsum_v, nsum_h.at[wid])


def _make_neg_kernel():
    mesh = plsc.VectorSubcoreMesh(core_axis_name="c", subcore_axis_name="s")
    return pl.kernel(
        _neg_body,
        out_type=jax.ShapeDtypeStruct((NW, BPW * EMBED), jnp.float32),
        mesh=mesh,
        scratch_types=(
            pltpu.VMEM((BPW * NEG,), jnp.int32),
            pltpu.VMEM((BPW * NEG,), jnp.int32),
            pltpu.VMEM((NROWS_BLK, 128), jnp.float32),
            pltpu.VMEM((NROWS_BLK, 128), jnp.float32),
            pltpu.VMEM((BPW * EMBED,), jnp.float32),
            pltpu.SemaphoreType.DMA,
        ),
        compiler_params=_SC_PARAMS,
    )


def _dot_body(wp_w, wp_u, isup_h, tsup_h, icb_h, tcb_h, nsum_h,
              pos_h, negl_h,
              isup_v, tsup_v, icb_v, tcb_v, nsum_v,
              emb0, ctx0, emb1, ctx1, pos_v, negl_v, sem):
    cid = lax.axis_index("c")
    sid = lax.axis_index("s")
    wid = sid * NUM_CORES + cid

    pltpu.sync_copy(isup_h.at[wid], isup_v)
    pltpu.sync_copy(tsup_h.at[wid], tsup_v)
    pltpu.sync_copy(icb_h.at[wid], icb_v)
    pltpu.sync_copy(tcb_h.at[wid], tcb_v)
    pltpu.sync_copy(nsum_h.at[wid], nsum_v)

    iota = lax.iota(jnp.int32, 16)
    bufs = ((emb0, ctx0), (emb1, ctx1))

    def issue(g, emb_b, ctx_b):
        pltpu.async_copy(wp_w.at[isup_v.at[pl.ds(g * BC, BC)]], emb_b, sem)
        pltpu.async_copy(wp_u.at[tsup_v.at[pl.ds(g * BC, BC)]], ctx_b, sem)

    def drain(emb_b, ctx_b):
        pltpu.make_async_copy(wp_w.at[pl.ds(0, BC)], emb_b, sem).wait()
        pltpu.make_async_copy(wp_u.at[pl.ds(0, BC)], ctx_b, sem).wait()

    def compute(blk, emb_b, ctx_b):
        cb_e = icb_v[pl.ds(blk * BC, 16)]
        cb_c = tcb_v[pl.ds(blk * BC, 16)]
        zero = jnp.zeros((16,), jnp.float32)

        @pl.loop(0, EMBED, init_carry=(zero, zero))
        def _dim(d, carry):
            acc_p, acc_n = carry
            e = plsc.load_gather(emb_b, [iota, cb_e + d])
            c = plsc.load_gather(ctx_b, [iota, cb_c + d])
            ns = nsum_v[pl.ds(d * BPW + blk * BC, 16)]
            return acc_p + e * c, acc_n + e * ns

        acc_p, acc_n = _dim
        pos_v[pl.ds(blk * BC, 16)] = acc_p
        negl_v[pl.ds(blk * BC, 16)] = -acc_n

    issue(0, *bufs[0])

    @pl.loop(0, NBLK, step=2)
    def _pair(blk):
        for b in (0, 1):
            g = blk + b

            @pl.when(g + 1 < NBLK)
            def _():
                issue(g + 1, *bufs[1 - b])

            drain(*bufs[b])
            compute(g, *bufs[b])

    pltpu.sync_copy(pos_v, pos_h.at[wid])
    pltpu.sync_copy(negl_v, negl_h.at[wid])


def _make_dot_kernel():
    mesh = plsc.VectorSubcoreMesh(core_axis_name="c", subcore_axis_name="s")
    return pl.kernel(
        _dot_body,
        out_type=(
            jax.ShapeDtypeStruct((NW, BPW), jnp.float32),
            jax.ShapeDtypeStruct((NW, BPW), jnp.float32),
        ),
        mesh=mesh,
        scratch_types=(
            pltpu.VMEM((BPW,), jnp.int32),
            pltpu.VMEM((BPW,), jnp.int32),
            pltpu.VMEM((BPW,), jnp.int32),
            pltpu.VMEM((BPW,), jnp.int32),
            pltpu.VMEM((BPW * EMBED,), jnp.float32),
            pltpu.VMEM((BC, 128), jnp.float32),
            pltpu.VMEM((BC, 128), jnp.float32),
            pltpu.VMEM((BC, 128), jnp.float32),
            pltpu.VMEM((BC, 128), jnp.float32),
            pltpu.VMEM((BPW,), jnp.float32),
            pltpu.VMEM((BPW,), jnp.float32),
            pltpu.SemaphoreType.DMA,
        ),
        compiler_params=_SC_PARAMS,
    )


def _loss_body(pos_ref, negl_ref, out_ref):
    def logsig(x):
        return jnp.minimum(x, 0.0) - jnp.log1p(jnp.exp(-jnp.abs(x)))

    total = jnp.sum(logsig(pos_ref[...])) + jnp.sum(logsig(negl_ref[...]))
    out_ref[0, 0] = -total / BATCH


def _split_idx(v):
    q = v // S
    return (v - q * S).astype(jnp.int32), (q * 32).astype(jnp.int32)


@jax.jit
def kernel(inputs, targets, neg_samples, W_w, W_u):
    wp_u = _pack_one(W_u)
    wp_w = _pack_one(W_w)

    isup, icb = _split_idx(inputs.astype(jnp.int32).reshape(BATCH))
    tsup, tcb = _split_idx(targets.astype(jnp.int32).reshape(BATCH))
    nsup, ncb = _split_idx(neg_samples.astype(jnp.int32))

    isup_h = isup.reshape(NW, BPW)
    tsup_h = tsup.reshape(NW, BPW)
    nsup_h = nsup.reshape(NW, BPW * NEG)
    icb_h = icb.reshape(NW, BPW)
    tcb_h = tcb.reshape(NW, BPW)
    ncb_h = ncb.reshape(NW, BPW, NEG).transpose(0, 2, 1).reshape(NW, BPW * NEG)

    nsum = _make_neg_kernel()(wp_u, nsup_h, ncb_h)
    pos, negl = _make_dot_kernel()(
        wp_w, wp_u, isup_h, tsup_h, icb_h, tcb_h, nsum)

    loss = pl.pallas_call(
        _loss_body,
        out_shape=jax.ShapeDtypeStruct((1, 1), jnp.float32),
        out_specs=pl.BlockSpec(memory_space=pltpu.SMEM),
    )(pos.reshape(128, 128), negl.reshape(128, 128))
    return loss[0, 0]


# speedup vs baseline: 1.0332x; 1.0030x over previous
import jax
import jax.numpy as jnp
from jax import lax
from jax.experimental import pallas as pl
from jax.experimental.pallas import tpu as pltpu
from jax.experimental.pallas import tpu_sc as plsc

VOCAB = 1000000
EMBED = 32
BATCH = 16384
NEG = 20

NUM_CORES = 2
NUM_SUBCORES = 16
NW = NUM_CORES * NUM_SUBCORES
BPW = BATCH // NW
BC = 16
NBLK = BPW // BC
NROWS_BLK = BC * NEG
NEG_CHUNKS = ((0, 128), (128, 128), (256, 64))

S = 250880
GW = 5120
SBG = S // GW
CLAMP_J = (VOCAB + GW - 1) // GW - 1


def _pack_body(w0, w1, w2, w3, u0, u1, u2, u3, ow, ou):
    ow[...] = jnp.concatenate(
        [w0[...], w1[...], w2[...], w3[...]], axis=0).T
    ou[...] = jnp.concatenate(
        [u0[...], u1[...], u2[...], u3[...]], axis=0).T


def _pack_tables(W_w, W_u):
    wt = W_w.T
    ut = W_u.T

    def in_spec(q):
        return pl.BlockSpec(
            (EMBED, GW),
            lambda i, q=q: (0, jnp.minimum(q * SBG + i, CLAMP_J)))

    out_spec = pl.BlockSpec((GW, 128), lambda i: (i, 0))
    return pl.pallas_call(
        _pack_body,
        grid=(SBG,),
        in_specs=[in_spec(q) for q in range(4)] * 2,
        out_specs=[out_spec, out_spec],
        out_shape=[jax.ShapeDtypeStruct((S, 128), jnp.float32)] * 2,
    )(wt, wt, wt, wt, ut, ut, ut, ut)


def _sc_body(wp_w, wp_u, isup_h, tsup_h, nsup_h, icb_h, tcb_h, ncb_h,
             pos_h, negl_h,
             isup_v, tsup_v, nsup_v, icb_v, tcb_v, ncb_v,
             emb0, ctx0, nrow0, emb1, ctx1, nrow1, pos_v, negl_v, sem):
    cid = lax.axis_index("c")
    sid = lax.axis_index("s")
    wid = sid * NUM_CORES + cid

    pltpu.sync_copy(isup_h.at[wid], isup_v)
    pltpu.sync_copy(tsup_h.at[wid], tsup_v)
    pltpu.sync_copy(nsup_h.at[wid], nsup_v)
    pltpu.sync_copy(icb_h.at[wid], icb_v)
    pltpu.sync_copy(tcb_h.at[wid], tcb_v)
    pltpu.sync_copy(ncb_h.at[wid], ncb_v)

    iota = lax.iota(jnp.int32, 16)
    nrows0 = iota * NEG
    bufs = ((emb0, ctx0, nrow0), (emb1, ctx1, nrow1))

    def issue(g, emb_b, ctx_b, nrow_b):
        pltpu.async_copy(wp_w.at[isup_v.at[pl.ds(g * BC, BC)]], emb_b, sem)
        pltpu.async_copy(wp_u.at[tsup_v.at[pl.ds(g * BC, BC)]], ctx_b, sem)
        for off, ln in NEG_CHUNKS:
            pltpu.async_copy(
                wp_u.at[nsup_v.at[pl.ds(g * NROWS_BLK + off, ln)]],
                nrow_b.at[pl.ds(off, ln)], sem)

    def drain(emb_b, ctx_b, nrow_b):
        pltpu.make_async_copy(wp_w.at[pl.ds(0, BC)], emb_b, sem).wait()
        pltpu.make_async_copy(wp_u.at[pl.ds(0, BC)], ctx_b, sem).wait()
        for off, ln in NEG_CHUNKS:
            pltpu.make_async_copy(wp_u.at[pl.ds(0, ln)],
                                  nrow_b.at[pl.ds(off, ln)],
                                  sem).wait()

    def compute(blk, emb_b, ctx_b, nrow_b):
        cb_e = icb_v[pl.ds(blk * BC, 16)]
        cb_c = tcb_v[pl.ds(blk * BC, 16)]
        zero = jnp.zeros((16,), jnp.float32)

        @pl.loop(0, EMBED, init_carry=(zero, zero))
        def _dim(d, carry):
            acc_p, acc_n = carry
            e = plsc.load_gather(emb_b, [iota, cb_e + d])
            c = plsc.load_gather(ctx_b, [iota, cb_c + d])
            s = jnp.zeros((16,), jnp.float32)
            for k in range(NEG):
                cb_nk = ncb_v[pl.ds(k * BPW + blk * BC, 16)]
                s = s + plsc.load_gather(nrow_b, [nrows0 + k, cb_nk + d])
            return acc_p + e * c, acc_n + e * s

        acc_p, acc_n = _dim
        pos_v[pl.ds(blk * BC, 16)] = acc_p
        negl_v[pl.ds(blk * BC, 16)] = -acc_n

    issue(0, *bufs[0])

    @pl.loop(0, NBLK, step=2)
    def _pair(blk):
        for b in (0, 1):
            g = blk + b

            @pl.when(g + 1 < NBLK)
            def _():
                issue(g + 1, *bufs[1 - b])

            drain(*bufs[b])
            compute(g, *bufs[b])

    pltpu.sync_copy(pos_v, pos_h.at[wid])
    pltpu.sync_copy(negl_v, negl_h.at[wid])


def _make_sc_kernel():
    mesh = plsc.VectorSubcoreMesh(core_axis_name="c", subcore_axis_name="s")
    return pl.kernel(
        _sc_body,
        out_type=(
            jax.ShapeDtypeStruct((NW, BPW), jnp.float32),
            jax.ShapeDtypeStruct((NW, BPW), jnp.float32),
        ),
        mesh=mesh,
        scratch_types=(
            pltpu.VMEM((BPW,), jnp.int32),
            pltpu.VMEM((BPW,), jnp.int32),
            pltpu.VMEM((BPW * NEG,), jnp.int32),
            pltpu.VMEM((BPW,), jnp.int32),
            pltpu.VMEM((BPW,), jnp.int32),
            pltpu.VMEM((BPW * NEG,), jnp.int32),
            pltpu.VMEM((BC, 128), jnp.float32),
            pltpu.VMEM((BC, 128), jnp.float32),
            pltpu.VMEM((NROWS_BLK, 128), jnp.float32),
            pltpu.VMEM((BC, 128), jnp.float32),
            pltpu.VMEM((BC, 128), jnp.float32),
            pltpu.VMEM((NROWS_BLK, 128), jnp.float32),
            pltpu.VMEM((BPW,), jnp.float32),
            pltpu.VMEM((BPW,), jnp.float32),
            pltpu.SemaphoreType.DMA,
        ),
        compiler_params=pltpu.CompilerParams(needs_layout_passes=False),
    )


def _loss_body(pos_ref, negl_ref, out_ref):
    def logsig(x):
        return jnp.minimum(x, 0.0) - jnp.log1p(jnp.exp(-jnp.abs(x)))

    total = jnp.sum(logsig(pos_ref[...])) + jnp.sum(logsig(negl_ref[...]))
    out_ref[0, 0] = -total / BATCH


def _split_idx(v):
    q = v // S
    return (v - q * S).astype(jnp.int32), (q * 32).astype(jnp.int32)


@jax.jit
def kernel(inputs, targets, neg_samples, W_w, W_u):
    wp_w, wp_u = _pack_tables(W_w, W_u)

    isup, icb = _split_idx(inputs.astype(jnp.int32).reshape(BATCH))
    tsup, tcb = _split_idx(targets.astype(jnp.int32).reshape(BATCH))
    nsup, ncb = _split_idx(neg_samples.astype(jnp.int32))

    isup_h = isup.reshape(NW, BPW)
    tsup_h = tsup.reshape(NW, BPW)
    nsup_h = nsup.reshape(NW, BPW * NEG)
    icb_h = icb.reshape(NW, BPW)
    tcb_h = tcb.reshape(NW, BPW)
    ncb_h = ncb.reshape(NW, BPW, NEG).transpose(0, 2, 1).reshape(NW, BPW * NEG)

    pos, negl = _make_sc_kernel()(
        wp_w, wp_u, isup_h, tsup_h, nsup_h, icb_h, tcb_h, ncb_h)

    loss = pl.pallas_call(
        _loss_body,
        out_shape=jax.ShapeDtypeStruct((1, 1), jnp.float32),
        out_specs=pl.BlockSpec(memory_space=pltpu.SMEM),
    )(pos.reshape(128, 128), negl.reshape(128, 128))
    return loss[0, 0]
